# baseline (speedup 1.0000x reference)
import dataclasses
import functools

import jax
import jax.numpy as jnp
from jax import lax
from jax.experimental import pallas as pl
from jax.experimental.pallas import tpu as pltpu
from jax.experimental.pallas import tpu_sc as plsc

N = 10000
E = 320000
NC = 2
NS = 16
NW = NC * NS
EPW = E // NW
CH = 125
NCH = EPW // CH
SHARE = N // NS

_sc_params = pltpu.CompilerParams(
    needs_layout_passes=False, use_tc_tiling_on_sc=False
)

_vmesh = plsc.VectorSubcoreMesh(
    core_axis_name="c", subcore_axis_name="s", num_cores=NC, num_subcores=NS
)


@functools.partial(
    pl.kernel,
    out_type=(
        jax.ShapeDtypeStruct((NW, N), jnp.float32),
        jax.ShapeDtypeStruct((NW, N), jnp.float32),
    ),
    mesh=_vmesh,
    scratch_types=[
        pltpu.VMEM((EPW,), jnp.int32),
        pltpu.VMEM((EPW,), jnp.int32),
        pltpu.VMEM((N,), jnp.float32),
        pltpu.VMEM((N,), jnp.float32),
        pltpu.SemaphoreType.DMA,
    ],
    compiler_params=_sc_params,
)
def _sc_degrees(src_hbm, dst_hbm, outs_hbm, outd_hbm, slab_s, slab_d, hist_s,
                hist_d, dsem):
    cid = lax.axis_index("c")
    sid = lax.axis_index("s")
    wid = cid * NS + sid
    ones = jnp.ones((16,), jnp.float32)
    zeros = jnp.zeros((16,), jnp.float32)

    cp_s = pltpu.async_copy(src_hbm.at[pl.ds(wid * EPW, EPW)], slab_s, dsem)
    cp_d = pltpu.async_copy(dst_hbm.at[pl.ds(wid * EPW, EPW)], slab_d, dsem)

    @pl.loop(0, N // 80)
    def _(i):
        for u in range(5):
            off = i * 80 + u * 16
            hist_s[pl.ds(off, 16)] = zeros
            hist_d[pl.ds(off, 16)] = zeros

    cp_s.wait()
    cp_d.wait()

    @pl.loop(0, EPW // 80)
    def _(g):
        for u in range(5):
            off = g * 80 + u * 16
            plsc.addupdate_scatter(hist_s, [slab_s[pl.ds(off, 16)]], ones)
            plsc.addupdate_scatter(hist_d, [slab_d[pl.ds(off, 16)]], ones)

    pltpu.sync_copy(hist_s, outs_hbm.at[wid])
    pltpu.sync_copy(hist_d, outd_hbm.at[wid])


K = 10
NG = NCH // K


def _make_sc_agg(F):
    @functools.partial(
        pl.kernel,
        out_type=jax.ShapeDtypeStruct((NC, N, F), jnp.float32),
        mesh=_vmesh,
        scratch_types=[
            pltpu.VMEM((NCH, CH), jnp.int32),
            pltpu.VMEM((NCH, CH), jnp.int32),
            pltpu.VMEM((2, K, CH, F), jnp.float32),
            pltpu.VMEM_SHARED((N, F), jnp.float32),
            pltpu.SemaphoreType.DMA,
            pltpu.SemaphoreType.DMA,
        ],
        compiler_params=_sc_params,
    )
    def _sc_agg(h_hbm, srcs_hbm, dsts_hbm, z_hbm, out_hbm, src_v, dst_v, rows_v,
                acc_sh, gsem, ssem):
        cid = lax.axis_index("c")
        sid = lax.axis_index("s")
        wid = cid * NS + sid

        def drain(sem):
            pltpu.make_async_copy(h_hbm.at[src_v.at[0]], rows_v.at[0, 0], sem).wait()

        pltpu.async_copy(srcs_hbm.at[wid], src_v, gsem)
        pltpu.async_copy(dsts_hbm.at[wid], dst_v, gsem)
        pltpu.make_async_copy(srcs_hbm.at[wid], src_v, gsem).wait()
        pltpu.make_async_copy(dsts_hbm.at[wid], dst_v, gsem).wait()

        @pl.loop(0, K)
        def _(b):
            pltpu.async_copy(h_hbm.at[src_v.at[b]], rows_v.at[0, b], gsem)

        pltpu.sync_copy(z_hbm.at[pl.ds(sid * SHARE, SHARE)],
                        acc_sh.at[pl.ds(sid * SHARE, SHARE)])
        plsc.subcore_barrier()

        @pl.loop(0, NG)
        def _(g):
            half = lax.rem(g, 2)
            other = 1 - half

            @pl.loop(0, K)
            def _(b):
                drain(gsem)

            @pl.when(g > 0)
            def _():
                @pl.loop(0, K)
                def _(b):
                    drain(ssem)

            @pl.when(g < NG - 1)
            def _():
                @pl.loop(0, K)
                def _(b):
                    pltpu.async_copy(h_hbm.at[src_v.at[(g + 1) * K + b]],
                                     rows_v.at[other, b], gsem)

            @pl.loop(0, K)
            def _(b):
                pltpu.async_copy(rows_v.at[half, b],
                                 acc_sh.at[dst_v.at[g * K + b]], ssem,
                                 add=True)

        @pl.loop(0, K)
        def _(b):
            drain(ssem)
        plsc.subcore_barrier()

        pltpu.sync_copy(acc_sh.at[pl.ds(sid * SHARE, SHARE)],
                        out_hbm.at[cid, pl.ds(sid * SHARE, SHARE)])

    return _sc_agg


_sc_agg32 = _make_sc_agg(32)
_sc_agg16 = _make_sc_agg(16)


def _norms_body(ds_ref, dd_ref, h1_ref, ns_ref, nd_ref, h1s_ref):
    s = jnp.sum(ds_ref[...], axis=0) + 1.0
    d = jnp.sum(dd_ref[...], axis=0) + 1.0
    ns = lax.rsqrt(s)[:, None]
    ns_ref[...] = ns
    nd_ref[...] = lax.rsqrt(d)[:, None]
    h1s_ref[...] = h1_ref[...] * ns


_tc_normscale = pl.pallas_call(
    _norms_body,
    out_shape=(
        jax.ShapeDtypeStruct((N, 1), jnp.float32),
        jax.ShapeDtypeStruct((N, 1), jnp.float32),
        jax.ShapeDtypeStruct((N, 32), jnp.float32),
    ),
)

_RB = 2000
_NRB = N // _RB


def _l1_body(x_ref, w_ref, o_ref):
    o_ref[...] = jnp.dot(x_ref[...], w_ref[...],
                         preferred_element_type=jnp.float32)


_tc_matmul1 = pl.pallas_call(
    _l1_body,
    grid=(_NRB,),
    in_specs=[
        pl.BlockSpec((_RB, 128), lambda i: (i, 0)),
        pl.BlockSpec((128, 32), lambda i: (0, 0)),
    ],
    out_specs=pl.BlockSpec((_RB, 32), lambda i: (i, 0)),
    out_shape=jax.ShapeDtypeStruct((N, 32), jnp.float32),
)


def _l2_body(p_ref, h1s_ref, nd_ref, b1_ref, w2_ref, ns_ref, o_ref):
    agg = p_ref[0] + p_ref[1] + h1s_ref[...]
    x2 = jnp.maximum(agg * nd_ref[...] + b1_ref[...], 0.0)
    h2 = jnp.dot(x2, w2_ref[...], preferred_element_type=jnp.float32)
    o_ref[...] = h2 * ns_ref[...]


_tc_layer2 = pl.pallas_call(
    _l2_body,
    grid=(_NRB,),
    in_specs=[
        pl.BlockSpec((NC, _RB, 32), lambda i: (0, i, 0)),
        pl.BlockSpec((_RB, 32), lambda i: (i, 0)),
        pl.BlockSpec((_RB, 1), lambda i: (i, 0)),
        pl.BlockSpec((1, 32), lambda i: (0, 0)),
        pl.BlockSpec((32, 16), lambda i: (0, 0)),
        pl.BlockSpec((_RB, 1), lambda i: (i, 0)),
    ],
    out_specs=pl.BlockSpec((_RB, 16), lambda i: (i, 0)),
    out_shape=jax.ShapeDtypeStruct((N, 16), jnp.float32),
)


def _out_body(p_ref, h2s_ref, nd_ref, b2_ref, o_ref):
    z = (p_ref[0] + p_ref[1] + h2s_ref[...]) * nd_ref[...] + b2_ref[...]
    m = jnp.max(z, axis=1, keepdims=True)
    e = jnp.exp(z - m)
    o_ref[...] = (z - m) - jnp.log(jnp.sum(e, axis=1, keepdims=True))


_tc_out = pl.pallas_call(
    _out_body,
    grid=(_NRB,),
    in_specs=[
        pl.BlockSpec((NC, _RB, 16), lambda i: (0, i, 0)),
        pl.BlockSpec((_RB, 16), lambda i: (i, 0)),
        pl.BlockSpec((_RB, 1), lambda i: (i, 0)),
        pl.BlockSpec((1, 16), lambda i: (0, 0)),
    ],
    out_specs=pl.BlockSpec((_RB, 16), lambda i: (i, 0)),
    out_shape=jax.ShapeDtypeStruct((N, 16), jnp.float32),
)


@jax.jit
def kernel(features, edge_index, W1, b1, W2, b2):
    src = edge_index[0]
    dst = edge_index[1]
    src_slab = src.reshape(NW, NCH, CH)
    dst_slab = dst.reshape(NW, NCH, CH)

    z32 = jnp.zeros((N, 32), jnp.float32)
    z16 = jnp.zeros((N, 16), jnp.float32)

    degs, degd = _sc_degrees(src, dst)
    h1 = _tc_matmul1(features, W1)
    norm_src, norm_dst, h1s = _tc_normscale(degs, degd, h1)
    p1 = _sc_agg32(h1s, src_slab, dst_slab, z32)
    h2s = _tc_layer2(p1, h1s, norm_dst, b1.reshape(1, 32), W2, norm_src)
    p2 = _sc_agg16(h2s, src_slab, dst_slab, z16)
    return _tc_out(p2, h2s, norm_dst, b2.reshape(1, 16))


# speedup vs baseline: 1.0012x; 1.0012x over previous
import functools

import jax
import jax.numpy as jnp
from jax import lax
from jax.experimental import pallas as pl
from jax.experimental.pallas import tpu as pltpu
from jax.experimental.pallas import tpu_sc as plsc

N = 10000
E = 320000
NC = 2
NS = 16
NW = NC * NS
EPW = E // NW
CH = 125
NCH = EPW // CH
SHARE = N // NS

_sc_params = pltpu.CompilerParams(
    needs_layout_passes=False, use_tc_tiling_on_sc=False
)

_vmesh = plsc.VectorSubcoreMesh(
    core_axis_name="c", subcore_axis_name="s", num_cores=NC, num_subcores=NS
)


@functools.partial(
    pl.kernel,
    out_type=(
        jax.ShapeDtypeStruct((NW, N), jnp.float32),
        jax.ShapeDtypeStruct((NW, N), jnp.float32),
    ),
    mesh=_vmesh,
    scratch_types=[
        pltpu.VMEM((EPW,), jnp.int32),
        pltpu.VMEM((EPW,), jnp.int32),
        pltpu.VMEM((N,), jnp.float32),
        pltpu.VMEM((N,), jnp.float32),
        pltpu.SemaphoreType.DMA,
    ],
    compiler_params=_sc_params,
)
def _sc_degrees(src_hbm, dst_hbm, outs_hbm, outd_hbm, slab_s, slab_d, hist_s,
                hist_d, dsem):
    cid = lax.axis_index("c")
    sid = lax.axis_index("s")
    wid = cid * NS + sid
    ones = jnp.ones((16,), jnp.float32)
    zeros = jnp.zeros((16,), jnp.float32)

    cp_s = pltpu.async_copy(src_hbm.at[pl.ds(wid * EPW, EPW)], slab_s, dsem)
    cp_d = pltpu.async_copy(dst_hbm.at[pl.ds(wid * EPW, EPW)], slab_d, dsem)

    @pl.loop(0, N // 80)
    def _(i):
        for u in range(5):
            off = i * 80 + u * 16
            hist_s[pl.ds(off, 16)] = zeros
            hist_d[pl.ds(off, 16)] = zeros

    cp_s.wait()
    cp_d.wait()

    @pl.loop(0, EPW // 80)
    def _(g):
        for u in range(5):
            off = g * 80 + u * 16
            plsc.addupdate_scatter(hist_s, [slab_s[pl.ds(off, 16)]], ones)
            plsc.addupdate_scatter(hist_d, [slab_d[pl.ds(off, 16)]], ones)

    pltpu.sync_copy(hist_s, outs_hbm.at[wid])
    pltpu.sync_copy(hist_d, outd_hbm.at[wid])


K = 10
NG = NCH // K


def _make_sc_agg(F):
    @functools.partial(
        pl.kernel,
        out_type=jax.ShapeDtypeStruct((NC, N, F), jnp.float32),
        mesh=_vmesh,
        scratch_types=[
            pltpu.VMEM((NCH, CH), jnp.int32),
            pltpu.VMEM((NCH, CH), jnp.int32),
            pltpu.VMEM((2, K, CH, F), jnp.float32),
            pltpu.VMEM_SHARED((N, F), jnp.float32),
            pltpu.SemaphoreType.DMA,
            pltpu.SemaphoreType.DMA,
        ],
        compiler_params=_sc_params,
    )
    def _sc_agg(h_hbm, srcs_hbm, dsts_hbm, z_hbm, out_hbm, src_v, dst_v, rows_v,
                acc_sh, gsem, ssem):
        cid = lax.axis_index("c")
        sid = lax.axis_index("s")
        wid = cid * NS + sid

        def drain(sem):
            pltpu.make_async_copy(h_hbm.at[src_v.at[0]], rows_v.at[0, 0], sem).wait()

        pltpu.async_copy(srcs_hbm.at[wid], src_v, gsem)
        pltpu.async_copy(dsts_hbm.at[wid], dst_v, gsem)
        pltpu.make_async_copy(srcs_hbm.at[wid], src_v, gsem).wait()
        pltpu.make_async_copy(dsts_hbm.at[wid], dst_v, gsem).wait()

        @pl.loop(0, K)
        def _(b):
            pltpu.async_copy(h_hbm.at[src_v.at[b]], rows_v.at[0, b], gsem)

        pltpu.sync_copy(z_hbm.at[pl.ds(sid * SHARE, SHARE)],
                        acc_sh.at[pl.ds(sid * SHARE, SHARE)])
        plsc.subcore_barrier()

        @pl.loop(0, NG)
        def _(g):
            half = lax.rem(g, 2)
            other = 1 - half

            @pl.loop(0, K)
            def _(b):
                drain(gsem)

            @pl.when(g > 0)
            def _():
                @pl.loop(0, K)
                def _(b):
                    drain(ssem)

            @pl.when(g < NG - 1)
            def _():
                @pl.loop(0, K)
                def _(b):
                    pltpu.async_copy(h_hbm.at[src_v.at[(g + 1) * K + b]],
                                     rows_v.at[other, b], gsem)

            @pl.loop(0, K)
            def _(b):
                pltpu.async_copy(rows_v.at[half, b],
                                 acc_sh.at[dst_v.at[g * K + b]], ssem,
                                 add=True)

        @pl.loop(0, K)
        def _(b):
            drain(ssem)
        plsc.subcore_barrier()

        pltpu.sync_copy(acc_sh.at[pl.ds(sid * SHARE, SHARE)],
                        out_hbm.at[cid, pl.ds(sid * SHARE, SHARE)])

    return _sc_agg


_sc_agg32 = _make_sc_agg(32)
_sc_agg16 = _make_sc_agg(16)


def _norms_body(ds_ref, dd_ref, h1_ref, ns_ref, nd_ref, h1s_ref):
    s = jnp.sum(ds_ref[...], axis=0) + 1.0
    d = jnp.sum(dd_ref[...], axis=0) + 1.0
    ns = lax.rsqrt(s)[:, None]
    ns_ref[...] = ns
    nd_ref[...] = lax.rsqrt(d)[:, None]
    h1s_ref[...] = h1_ref[...] * ns


_tc_normscale = pl.pallas_call(
    _norms_body,
    out_shape=(
        jax.ShapeDtypeStruct((N, 1), jnp.float32),
        jax.ShapeDtypeStruct((N, 1), jnp.float32),
        jax.ShapeDtypeStruct((N, 32), jnp.float32),
    ),
)

_RB = 2000
_NRB = N // _RB


def _l1_body(x_ref, w_ref, o_ref):
    o_ref[...] = jnp.dot(x_ref[...], w_ref[...],
                         preferred_element_type=jnp.float32)


_tc_matmul1 = pl.pallas_call(
    _l1_body,
    grid=(_NRB,),
    in_specs=[
        pl.BlockSpec((_RB, 128), lambda i: (i, 0)),
        pl.BlockSpec((128, 32), lambda i: (0, 0)),
    ],
    out_specs=pl.BlockSpec((_RB, 32), lambda i: (i, 0)),
    out_shape=jax.ShapeDtypeStruct((N, 32), jnp.float32),
)


def _l2_body(p_ref, h1s_ref, nd_ref, b1_ref, w2_ref, ns_ref, o_ref):
    agg = p_ref[0] + p_ref[1] + h1s_ref[...]
    x2 = jnp.maximum(agg * nd_ref[...] + b1_ref[...], 0.0)
    h2 = jnp.dot(x2, w2_ref[...], preferred_element_type=jnp.float32)
    o_ref[...] = h2 * ns_ref[...]


_tc_layer2 = pl.pallas_call(
    _l2_body,
    grid=(_NRB,),
    in_specs=[
        pl.BlockSpec((NC, _RB, 32), lambda i: (0, i, 0)),
        pl.BlockSpec((_RB, 32), lambda i: (i, 0)),
        pl.BlockSpec((_RB, 1), lambda i: (i, 0)),
        pl.BlockSpec((1, 32), lambda i: (0, 0)),
        pl.BlockSpec((32, 16), lambda i: (0, 0)),
        pl.BlockSpec((_RB, 1), lambda i: (i, 0)),
    ],
    out_specs=pl.BlockSpec((_RB, 16), lambda i: (i, 0)),
    out_shape=jax.ShapeDtypeStruct((N, 16), jnp.float32),
)


def _out_body(p_ref, h2s_ref, nd_ref, b2_ref, o_ref):
    z = (p_ref[0] + p_ref[1] + h2s_ref[...]) * nd_ref[...] + b2_ref[...]
    m = jnp.max(z, axis=1, keepdims=True)
    e = jnp.exp(z - m)
    o_ref[...] = (z - m) - jnp.log(jnp.sum(e, axis=1, keepdims=True))


_tc_out = pl.pallas_call(
    _out_body,
    grid=(_NRB,),
    in_specs=[
        pl.BlockSpec((NC, _RB, 16), lambda i: (0, i, 0)),
        pl.BlockSpec((_RB, 16), lambda i: (i, 0)),
        pl.BlockSpec((_RB, 1), lambda i: (i, 0)),
        pl.BlockSpec((1, 16), lambda i: (0, 0)),
    ],
    out_specs=pl.BlockSpec((_RB, 16), lambda i: (i, 0)),
    out_shape=jax.ShapeDtypeStruct((N, 16), jnp.float32),
)


@jax.jit
def kernel(features, edge_index, W1, b1, W2, b2):
    src = edge_index[0]
    dst = edge_index[1]
    src_slab = src.reshape(NW, NCH, CH)
    dst_slab = dst.reshape(NW, NCH, CH)

    z32 = jnp.zeros((N, 32), jnp.float32)
    z16 = jnp.zeros((N, 16), jnp.float32)

    degs, degd = _sc_degrees(src, dst)
    h1 = _tc_matmul1(features, W1)
    norm_src, norm_dst, h1s = _tc_normscale(degs, degd, h1)
    p1 = _sc_agg32(h1s, src_slab, dst_slab, z32)
    h2s = _tc_layer2(p1, h1s, norm_dst, b1.reshape(1, 32), W2, norm_src)
    p2 = _sc_agg16(h2s, src_slab, dst_slab, z16)
    return _tc_out(p2, h2s, norm_dst, b2.reshape(1, 16))
